# baseline (speedup 1.0000x reference)
import functools

import jax
import jax.numpy as jnp
from jax import lax
from jax.experimental import pallas as pl
from jax.experimental.pallas import tpu as pltpu
from jax.experimental.pallas import tpu_sc as plsc

N_FIELDS = 26
VOCAB = 100000
EMBED = 32
BATCH = 16384

_info = plsc.get_sparse_core_info()
_NC, _NS = _info.num_cores, _info.num_subcores
_NW = _NC * _NS
_BPW = BATCH // _NW


@functools.partial(
    pl.kernel,
    out_type=jax.ShapeDtypeStruct((N_FIELDS, BATCH, EMBED), jnp.float32),
    mesh=plsc.VectorSubcoreMesh(core_axis_name="c", subcore_axis_name="s"),
    scratch_types=[
        pltpu.VMEM((_BPW,), jnp.int32),
        pltpu.VMEM((_BPW, EMBED), jnp.float32),
        pltpu.SemaphoreType.DMA,
    ],
    compiler_params=pltpu.CompilerParams(use_tc_tiling_on_sc=False),
)
def _gather_all_fields(x_hbm, w_hbm, out_hbm, idx_v, rows_v, sem):
    wid = lax.axis_index("s") * _NC + lax.axis_index("c")
    base = wid * _BPW

    def body(i, carry):
        pltpu.sync_copy(x_hbm.at[i, pl.ds(base, _BPW)], idx_v)
        pltpu.async_copy(w_hbm.at[i].at[idx_v], rows_v, sem).wait()
        pltpu.sync_copy(rows_v, out_hbm.at[i, pl.ds(base, _BPW)])
        return carry

    lax.fori_loop(0, N_FIELDS, body, 0)


def kernel(x, W):
    xT = jnp.transpose(x.astype(jnp.int32))
    out = _gather_all_fields(xT, W)
    return tuple(out[i] for i in range(N_FIELDS))


# speedup vs baseline: 1.5546x; 1.5546x over previous
import functools

import jax
import jax.numpy as jnp
from jax import lax
from jax.experimental import pallas as pl
from jax.experimental.pallas import tpu as pltpu
from jax.experimental.pallas import tpu_sc as plsc

N_FIELDS = 26
VOCAB = 100000
EMBED = 32
BATCH = 16384

_info = plsc.get_sparse_core_info()
_NC, _NS = _info.num_cores, _info.num_subcores
_NW = _NC * _NS
_BPW = BATCH // _NW

_VBLK = 512
_VPAD = ((VOCAB + 1 + _VBLK - 1) // _VBLK) * _VBLK
_NROW32 = N_FIELDS * _VPAD


def _repack_block(x_ref, o_ref):
    o_ref[0] = jnp.concatenate(
        [jnp.swapaxes(x_ref[0, :, d * 128:(d + 1) * 128], 0, 1) for d in range(4)],
        axis=1,
    )


_tc_repack = pl.pallas_call(
    _repack_block,
    grid=(N_FIELDS, _VPAD // _VBLK),
    in_specs=[pl.BlockSpec((1, EMBED, _VBLK), lambda i, j: (i, 0, j))],
    out_specs=pl.BlockSpec((1, _VBLK // 4, 4 * EMBED), lambda i, j: (i, j, 0)),
    out_shape=jax.ShapeDtypeStruct((N_FIELDS, _VPAD // 4, 4 * EMBED), jnp.float32),
)


@functools.partial(
    pl.kernel,
    out_type=jax.ShapeDtypeStruct((N_FIELDS, BATCH, EMBED), jnp.float32),
    mesh=plsc.VectorSubcoreMesh(core_axis_name="c", subcore_axis_name="s"),
    scratch_types=[
        pltpu.VMEM((N_FIELDS, _BPW), jnp.int32),
        pltpu.VMEM((_BPW,), jnp.int32),
        pltpu.VMEM((_BPW, EMBED), jnp.float32),
        pltpu.SemaphoreType.DMA,
    ],
    compiler_params=pltpu.CompilerParams(use_tc_tiling_on_sc=False),
)
def _gather_all_fields(x_hbm, w_hbm, out_hbm, idx_all, idx_v, rows_v, sem):
    wid = lax.axis_index("s") * _NC + lax.axis_index("c")
    base = wid * _BPW

    pltpu.sync_copy(x_hbm.at[:, pl.ds(base, _BPW)], idx_all)

    def field_body(i, carry):
        tbase = i * _VPAD

        def remap(s, c2):
            v = idx_all[i, pl.ds(s * 16, 16)]
            r = ((v >> 9) << 9) + ((v & 127) << 2) + ((v >> 7) & 3)
            idx_v[pl.ds(s * 16, 16)] = r + tbase
            return c2

        lax.fori_loop(0, _BPW // 16, remap, 0)
        pltpu.async_copy(w_hbm.at[idx_v], rows_v, sem).wait()
        pltpu.sync_copy(rows_v, out_hbm.at[i, pl.ds(base, _BPW)])
        return carry

    lax.fori_loop(0, N_FIELDS, field_body, 0)


def kernel(x, W):
    wt = jnp.transpose(W, (0, 2, 1))
    w_pk = _tc_repack(wt).reshape(_NROW32, EMBED)
    xt = jnp.transpose(x.astype(jnp.int32))
    out = _gather_all_fields(xt, w_pk)
    return tuple(out[i] for i in range(N_FIELDS))


# speedup vs baseline: 3.6622x; 2.3558x over previous
import functools

import jax
import jax.numpy as jnp
from jax import lax
from jax.experimental import pallas as pl
from jax.experimental.pallas import tpu as pltpu
from jax.experimental.pallas import tpu_sc as plsc

N_FIELDS = 26
VOCAB = 100000
EMBED = 32
BATCH = 16384

_info = plsc.get_sparse_core_info()
_NC, _NS = _info.num_cores, _info.num_subcores
_NW = _NC * _NS
_BPW = BATCH // _NW

_VBLK = 2048
_VPAD = ((VOCAB + 1 + _VBLK - 1) // _VBLK) * _VBLK
_NROW32 = N_FIELDS * _VPAD


def _repack_block(x_ref, o_ref):
    eye = jnp.eye(128, dtype=jnp.float32)
    x = x_ref[0]
    groups = []
    for s in range(_VBLK // 512):
        pieces = [
            jax.lax.dot_general(
                eye,
                x[:, s * 512 + d * 128:s * 512 + (d + 1) * 128],
                ((( 1,), (1,)), ((), ())),
                preferred_element_type=jnp.float32,
            )
            for d in range(4)
        ]
        groups.append(jnp.concatenate(pieces, axis=1))
    o_ref[0] = jnp.concatenate(groups, axis=0)


_tc_repack = pl.pallas_call(
    _repack_block,
    grid=(N_FIELDS, _VPAD // _VBLK),
    in_specs=[pl.BlockSpec((1, EMBED, _VBLK), lambda i, j: (i, 0, j))],
    out_specs=pl.BlockSpec((1, _VBLK // 4, 4 * EMBED), lambda i, j: (i, j, 0)),
    out_shape=jax.ShapeDtypeStruct((N_FIELDS, _VPAD // 4, 4 * EMBED), jnp.float32),
)


@functools.partial(
    pl.kernel,
    out_type=jax.ShapeDtypeStruct((N_FIELDS, BATCH, EMBED), jnp.float32),
    mesh=plsc.VectorSubcoreMesh(core_axis_name="c", subcore_axis_name="s"),
    scratch_types=[
        pltpu.VMEM((N_FIELDS, _BPW), jnp.int32),
        pltpu.VMEM((_BPW,), jnp.int32),
        pltpu.VMEM((_BPW, EMBED), jnp.float32),
        pltpu.SemaphoreType.DMA,
    ],
    compiler_params=pltpu.CompilerParams(use_tc_tiling_on_sc=False),
)
def _gather_all_fields(x_hbm, w_hbm, out_hbm, idx_all, idx_v, rows_v, sem):
    wid = lax.axis_index("s") * _NC + lax.axis_index("c")
    base = wid * _BPW

    pltpu.sync_copy(x_hbm.at[:, pl.ds(base, _BPW)], idx_all)

    def field_body(i, carry):
        tbase = i * _VPAD

        def remap(s, c2):
            v = idx_all[i, pl.ds(s * 16, 16)]
            r = ((v >> 9) << 9) + ((v & 127) << 2) + ((v >> 7) & 3)
            idx_v[pl.ds(s * 16, 16)] = r + tbase
            return c2

        lax.fori_loop(0, _BPW // 16, remap, 0)
        pltpu.async_copy(w_hbm.at[idx_v], rows_v, sem).wait()
        pltpu.sync_copy(rows_v, out_hbm.at[i, pl.ds(base, _BPW)])
        return carry

    lax.fori_loop(0, N_FIELDS, field_body, 0)


def kernel(x, W):
    wt = jnp.transpose(W, (0, 2, 1))
    w_pk = _tc_repack(wt).reshape(_NROW32, EMBED)
    xt = jnp.transpose(x.astype(jnp.int32))
    out = _gather_all_fields(xt, w_pk)
    return tuple(out[i] for i in range(N_FIELDS))
